# baseline (speedup 1.0000x reference)
import jax
import jax.numpy as jnp
from jax import lax
from jax.experimental import pallas as pl
from jax.experimental.pallas import tpu as pltpu
from jax.experimental.pallas import tpu_sc as plsc

B, C, H, W = 2, 96, 224, 224
OH, OW = 448, 448
L = 16
NC, NS = 2, 16
NW = NC * NS
SLICES = B * C
SPT = SLICES // NW
IRW = 16
ORW = 32
NIB = 7
NWIN = OH // ORW
NQ = SPT * NWIN
GROUPS = W // L


def _sc_body(in_hbm, jit_hbm, out_hbm, jbuf, *rest):
    inbs = rest[0:NIB]
    outbs = rest[NIB:NIB + 2]
    sins = rest[NIB + 2:2 * NIB + 2]
    souts = rest[2 * NIB + 2:2 * NIB + 4]
    wid = lax.axis_index("s") * NC + lax.axis_index("c")
    s0 = wid * SPT
    b = s0 // C

    pltpu.sync_copy(jit_hbm, jbuf)
    jy0 = jbuf[pl.ds(0, L)]
    jx0 = jbuf[pl.ds(L, L)]
    jy1 = jbuf[pl.ds(2 * L, L)]
    jx1 = jbuf[pl.ds(3 * L, L)]
    bv = jnp.broadcast_to(b, (L,))
    jy = jnp.where(bv == 0, jy0, jy1)
    jx = jnp.where(bv == 0, jx0, jx1)
    half = jnp.full((L,), 0.5, jnp.float32)
    one = jnp.full((L,), 1, jnp.int32)
    two = jnp.full((L,), 2, jnp.int32)
    dyv = jnp.where(jy >= half, two, one)
    dxv = jnp.where(jx >= half, two, one)

    iota = lax.iota(jnp.int32, L)
    wmax = jnp.full((L,), OW - 1, jnp.int32)
    dy_is1 = dyv == one
    dy_is2 = dyv == two
    colv = [jnp.minimum(2 * L * g + 2 * iota + dxv, wmax) for g in range(GROUPS)]
    r0 = jnp.full((L,), 0, jnp.int32)
    rtop = jnp.full((L,), ORW - 1, jnp.int32)

    zf = jnp.zeros((L,), jnp.float32)

    for g in range(OW // L):
        def zb(i, _, g=g):
            outbs[0][i, pl.ds(g * L, L)] = zf
            outbs[1][i, pl.ds(g * L, L)] = zf
            return 0
        lax.fori_loop(0, ORW, zb, 0)

    def in_copy_for(si, k):
        return pltpu.make_async_copy(
            in_hbm.at[s0 + si, pl.ds(IRW * k, IRW), :], inbs[k % NIB],
            sins[k % NIB])

    def out_copy_for(si, k):
        return pltpu.make_async_copy(
            outbs[k % 2], out_hbm.at[s0 + si, pl.ds(ORW * k, ORW), :],
            souts[k % 2])

    def plane(si, _):
        def in_copy(k):
            return in_copy_for(si, k)

        def out_copy(k):
            return out_copy_for(si, k)

        for k in range(NWIN):
            inb = inbs[k % NIB]
            prev = inbs[(k - 1) % NIB]
            outb = outbs[k % 2]

            in_copy(k).wait()
            if k + 1 < NWIN:
                in_copy(k + 1).start()
            if k >= 2:
                out_copy(k - 2).wait()
            elif k == 0:
                @pl.when(si > 0)
                def _():
                    out_copy_for(si - 1, NWIN - 2).wait()
            else:
                @pl.when(si > 0)
                def _():
                    out_copy_for(si - 1, NWIN - 1).wait()

            def rz(g, _, outb=outb, k=k):
                outb[ORW - 1, pl.ds(g * L, L)] = zf
                if k == 0:
                    outb[0, pl.ds(g * L, L)] = zf
                return 0
            lax.fori_loop(0, OW // L, rz, 0)

            if k > 0:
                for g in range(GROUPS):
                    plsc.store_scatter(outb, [r0, colv[g]],
                                       prev[IRW - 1, pl.ds(g * L, L)],
                                       mask=dy_is2)
            m63 = None if k == NWIN - 1 else dy_is1
            for g in range(GROUPS):
                plsc.store_scatter(outb, [rtop, colv[g]],
                                   inb[IRW - 1, pl.ds(g * L, L)], mask=m63)

            @plsc.parallel_loop(0, IRW - 1, unroll=8)
            def row(j, inb=inb, outb=outb):
                rlv = 2 * j + dyv
                for g in range(GROUPS):
                    plsc.store_scatter(outb, [rlv, colv[g]],
                                       inb[j, pl.ds(g * L, L)])

            out_copy(k).start()
            if k == NWIN - 1:
                @pl.when(si < SPT - 1)
                def _():
                    in_copy_for(si + 1, 0).start()
        return 0

    in_copy_for(0, 0).start()
    lax.fori_loop(0, SPT, plane, 0)
    out_copy_for(SPT - 1, NWIN - 2).wait()
    out_copy_for(SPT - 1, NWIN - 1).wait()


@jax.jit
def _zero_upsample_sc(in3, jit64):
    mesh = plsc.VectorSubcoreMesh(core_axis_name="c", subcore_axis_name="s")
    f = pl.kernel(
        _sc_body,
        out_type=jax.ShapeDtypeStruct((SLICES, OH, OW), jnp.float32),
        mesh=mesh,
        compiler_params=pltpu.CompilerParams(needs_layout_passes=False),
        scratch_types=(
            [pltpu.VMEM((4 * L,), jnp.float32)]
            + [pltpu.VMEM((IRW, W), jnp.float32)] * NIB
            + [pltpu.VMEM((ORW, OW), jnp.float32)] * 2
            + [pltpu.SemaphoreType.DMA] * (NIB + 2)
        ),
    )
    return f(in3, jit64)


def kernel(ten_in, jitter):
    jit64 = jnp.broadcast_to(
        jitter.astype(jnp.float32).reshape(4, 1), (4, L)).reshape(4 * L)
    out3 = _zero_upsample_sc(ten_in.reshape(SLICES, H, W), jit64)
    return out3.reshape(B, C, OH, OW)


# speedup vs baseline: 1.1451x; 1.1451x over previous
import jax
import jax.numpy as jnp
from jax import lax
from jax.experimental import pallas as pl
from jax.experimental.pallas import tpu as pltpu
from jax.experimental.pallas import tpu_sc as plsc

B, C, H, W = 2, 96, 224, 224
OH, OW = 448, 448
L = 16
NC, NS = 2, 16
NW = NC * NS
SLICES = B * C
SPT = SLICES // NW
IRW = 32
ORW = 64
NIB = 7
NWIN = OH // ORW
NQ = SPT * NWIN
GROUPS = W // L


def _sc_body(in_hbm, jit_hbm, out_hbm, jbuf, *rest):
    inbs = rest[0:NIB]
    outbs = rest[NIB:NIB + 2]
    sins = rest[NIB + 2:2 * NIB + 2]
    souts = rest[2 * NIB + 2:2 * NIB + 4]
    wid = lax.axis_index("s") * NC + lax.axis_index("c")
    s0 = wid * SPT
    b = s0 // C

    pltpu.sync_copy(jit_hbm, jbuf)
    jy0 = jbuf[pl.ds(0, L)]
    jx0 = jbuf[pl.ds(L, L)]
    jy1 = jbuf[pl.ds(2 * L, L)]
    jx1 = jbuf[pl.ds(3 * L, L)]
    bv = jnp.broadcast_to(b, (L,))
    jy = jnp.where(bv == 0, jy0, jy1)
    jx = jnp.where(bv == 0, jx0, jx1)
    half = jnp.full((L,), 0.5, jnp.float32)
    one = jnp.full((L,), 1, jnp.int32)
    two = jnp.full((L,), 2, jnp.int32)
    dyv = jnp.where(jy >= half, two, one)
    dxv = jnp.where(jx >= half, two, one)

    iota = lax.iota(jnp.int32, L)
    wmax = jnp.full((L,), OW - 1, jnp.int32)
    dy_is1 = dyv == one
    dy_is2 = dyv == two
    colv = [jnp.minimum(2 * L * g + 2 * iota + dxv, wmax) for g in range(GROUPS)]
    r0 = jnp.full((L,), 0, jnp.int32)
    rtop = jnp.full((L,), ORW - 1, jnp.int32)

    zf = jnp.zeros((L,), jnp.float32)

    for g in range(OW // L):
        def zb(i, _, g=g):
            outbs[0][i, pl.ds(g * L, L)] = zf
            outbs[1][i, pl.ds(g * L, L)] = zf
            return 0
        lax.fori_loop(0, ORW, zb, 0)

    def in_copy_for(si, k):
        return pltpu.make_async_copy(
            in_hbm.at[s0 + si, pl.ds(IRW * k, IRW), :], inbs[k % NIB],
            sins[k % NIB])

    def out_copy_for(si, k, buf):
        return pltpu.make_async_copy(
            outbs[buf], out_hbm.at[s0 + si, pl.ds(ORW * k, ORW), :],
            souts[buf])

    def pair(pr, _):
        for t in range(2 * NWIN):
            p, k = t // NWIN, t % NWIN
            si = 2 * pr + p
            inb = inbs[t % NIB]
            prev = inbs[(t - 1) % NIB]
            outb = outbs[t % 2]

            in_copy_for(si, k).wait()
            if t + 1 < 2 * NWIN:
                tn = t + 1
                in_copy_for(2 * pr + tn // NWIN, tn % NWIN).start()
            else:
                @pl.when(pr < SPT // 2 - 1)
                def _():
                    in_copy_for(2 * pr + 2, 0).start()
            if t >= 2:
                tp = t - 2
                out_copy_for(2 * pr + tp // NWIN, tp % NWIN, tp % 2).wait()
            else:
                tp = 2 * NWIN + t - 2
                @pl.when(pr > 0)
                def _():
                    out_copy_for(2 * pr - 2 + tp // NWIN, tp % NWIN,
                                 tp % 2).wait()

            def rz(g, _, outb=outb, k=k):
                outb[ORW - 1, pl.ds(g * L, L)] = zf
                if k == 0:
                    outb[0, pl.ds(g * L, L)] = zf
                return 0
            lax.fori_loop(0, OW // L, rz, 0)

            if k > 0:
                for g in range(GROUPS):
                    plsc.store_scatter(outb, [r0, colv[g]],
                                       prev[IRW - 1, pl.ds(g * L, L)],
                                       mask=dy_is2)
            mtop = None if k == NWIN - 1 else dy_is1
            for g in range(GROUPS):
                plsc.store_scatter(outb, [rtop, colv[g]],
                                   inb[IRW - 1, pl.ds(g * L, L)], mask=mtop)

            @plsc.parallel_loop(0, IRW - 1, unroll=8)
            def row(j, inb=inb, outb=outb):
                rlv = 2 * j + dyv
                for g in range(GROUPS):
                    plsc.store_scatter(outb, [rlv, colv[g]],
                                       inb[j, pl.ds(g * L, L)])

            out_copy_for(si, k, t % 2).start()
        return 0

    in_copy_for(0, 0).start()
    lax.fori_loop(0, SPT // 2, pair, 0)
    out_copy_for(SPT - 1, NWIN - 2, (2 * NWIN - 2) % 2).wait()
    out_copy_for(SPT - 1, NWIN - 1, (2 * NWIN - 1) % 2).wait()


@jax.jit
def _zero_upsample_sc(in3, jit64):
    mesh = plsc.VectorSubcoreMesh(core_axis_name="c", subcore_axis_name="s")
    f = pl.kernel(
        _sc_body,
        out_type=jax.ShapeDtypeStruct((SLICES, OH, OW), jnp.float32),
        mesh=mesh,
        compiler_params=pltpu.CompilerParams(needs_layout_passes=False),
        scratch_types=(
            [pltpu.VMEM((4 * L,), jnp.float32)]
            + [pltpu.VMEM((IRW, W), jnp.float32)] * NIB
            + [pltpu.VMEM((ORW, OW), jnp.float32)] * 2
            + [pltpu.SemaphoreType.DMA] * (NIB + 2)
        ),
    )
    return f(in3, jit64)


def kernel(ten_in, jitter):
    jit64 = jnp.broadcast_to(
        jitter.astype(jnp.float32).reshape(4, 1), (4, L)).reshape(4 * L)
    out3 = _zero_upsample_sc(ten_in.reshape(SLICES, H, W), jit64)
    return out3.reshape(B, C, OH, OW)


# speedup vs baseline: 1.2143x; 1.0605x over previous
import jax
import jax.numpy as jnp
from jax import lax
from jax.experimental import pallas as pl
from jax.experimental.pallas import tpu as pltpu
from jax.experimental.pallas import tpu_sc as plsc

B, C, H, W = 2, 96, 224, 224
OH, OW = 448, 448
L = 16
NC, NS = 2, 16
NW = NC * NS
SLICES = B * C
SPT = SLICES // NW
IRW = 32
ORW = 64
NWIN = OH // ORW
NQ = SPT * NWIN
GROUPS = W // L


def _sc_body(in_hbm, jit_hbm, out_hbm, jbuf, inb0, inb1, inb2,
             outb0, outb1, s_in0, s_in1, s_in2, s_out0, s_out1):
    wid = lax.axis_index("s") * NC + lax.axis_index("c")
    s0 = wid * SPT
    b = s0 // C

    pltpu.sync_copy(jit_hbm, jbuf)
    jy0 = jbuf[pl.ds(0, L)]
    jx0 = jbuf[pl.ds(L, L)]
    jy1 = jbuf[pl.ds(2 * L, L)]
    jx1 = jbuf[pl.ds(3 * L, L)]
    bv = jnp.broadcast_to(b, (L,))
    jy = jnp.where(bv == 0, jy0, jy1)
    jx = jnp.where(bv == 0, jx0, jx1)
    half = jnp.full((L,), 0.5, jnp.float32)
    one = jnp.full((L,), 1, jnp.int32)
    two = jnp.full((L,), 2, jnp.int32)
    dyv = jnp.where(jy >= half, two, one)
    dxv = jnp.where(jx >= half, two, one)

    iota = lax.iota(jnp.int32, L)
    wmax = jnp.full((L,), OW - 1, jnp.int32)
    dy_is1 = dyv == one
    dy_is2 = dyv == two
    colv = [jnp.minimum(2 * L * g + 2 * iota + dxv, wmax) for g in range(GROUPS)]
    r0 = jnp.full((L,), 0, jnp.int32)
    r63 = jnp.full((L,), ORW - 1, jnp.int32)

    zf = jnp.zeros((L,), jnp.float32)

    for g in range(OW // L):
        def zb(i, _, g=g):
            outb0[i, pl.ds(g * L, L)] = zf
            outb1[i, pl.ds(g * L, L)] = zf
            return 0
        lax.fori_loop(0, ORW, zb, 0)

    inbs = (inb0, inb1, inb2)
    outbs = (outb0, outb1)
    sins = (s_in0, s_in1, s_in2)
    souts = (s_out0, s_out1)

    def plane(si, _):
        def in_copy(k):
            return pltpu.make_async_copy(
                in_hbm.at[s0 + si, pl.ds(IRW * k, IRW), :], inbs[k % 3],
                sins[k % 3])

        def out_copy(k):
            return pltpu.make_async_copy(
                outbs[k % 2], out_hbm.at[s0 + si, pl.ds(ORW * k, ORW), :],
                souts[k % 2])

        in_copy(0).start()
        for k in range(NWIN):
            inb = inbs[k % 3]
            prev = inbs[(k - 1) % 3]
            outb = outbs[k % 2]

            in_copy(k).wait()
            if k + 1 < NWIN:
                in_copy(k + 1).start()
            if k >= 2:
                out_copy(k - 2).wait()

            def rz(g, _, outb=outb, k=k):
                outb[ORW - 1, pl.ds(g * L, L)] = zf
                if k == 0:
                    outb[0, pl.ds(g * L, L)] = zf
                return 0
            lax.fori_loop(0, OW // L, rz, 0)

            if k > 0:
                for g in range(GROUPS):
                    plsc.store_scatter(outb, [r0, colv[g]],
                                       prev[IRW - 1, pl.ds(g * L, L)],
                                       mask=dy_is2)
            m63 = None if k == NWIN - 1 else dy_is1
            for g in range(GROUPS):
                plsc.store_scatter(outb, [r63, colv[g]],
                                   inb[IRW - 1, pl.ds(g * L, L)], mask=m63)

            @plsc.parallel_loop(0, IRW - 1, unroll=4)
            def row(j, inb=inb, outb=outb):
                rlv = 2 * j + dyv
                for g in range(GROUPS):
                    plsc.store_scatter(outb, [rlv, colv[g]],
                                       inb[j, pl.ds(g * L, L)])

            out_copy(k).start()
        out_copy(NWIN - 2).wait()
        out_copy(NWIN - 1).wait()
        return 0

    lax.fori_loop(0, SPT, plane, 0)


@jax.jit
def _zero_upsample_sc(in3, jit64):
    mesh = plsc.VectorSubcoreMesh(core_axis_name="c", subcore_axis_name="s")
    f = pl.kernel(
        _sc_body,
        out_type=jax.ShapeDtypeStruct((SLICES, OH, OW), jnp.float32),
        mesh=mesh,
        compiler_params=pltpu.CompilerParams(needs_layout_passes=False),
        scratch_types=[
            pltpu.VMEM((4 * L,), jnp.float32),
            pltpu.VMEM((IRW, W), jnp.float32),
            pltpu.VMEM((IRW, W), jnp.float32),
            pltpu.VMEM((IRW, W), jnp.float32),
            pltpu.VMEM((ORW, OW), jnp.float32),
            pltpu.VMEM((ORW, OW), jnp.float32),
            pltpu.SemaphoreType.DMA,
            pltpu.SemaphoreType.DMA,
            pltpu.SemaphoreType.DMA,
            pltpu.SemaphoreType.DMA,
            pltpu.SemaphoreType.DMA,
        ],
    )
    return f(in3, jit64)


def kernel(ten_in, jitter):
    jit64 = jnp.broadcast_to(
        jitter.astype(jnp.float32).reshape(4, 1), (4, L)).reshape(4 * L)
    out3 = _zero_upsample_sc(ten_in.reshape(SLICES, H, W), jit64)
    return out3.reshape(B, C, OH, OW)
